# baseline (speedup 1.0000x reference)
import functools

import jax
import jax.numpy as jnp
from jax import lax
from jax.experimental import pallas as pl
from jax.experimental.pallas import tpu as pltpu
from jax.experimental.pallas import tpu_sc as plsc

_N_NODES = 10000
_N_EDGES = 320000
_D = 128

_NC = 2
_NS = 16
_L = 16
_NW = _NC * _NS
_E_W = _N_EDGES // _NW
_B = 200
_BPAD = 208
_NCHUNK = _E_W // _B
_G = _BPAD // _L

_mesh = plsc.VectorSubcoreMesh(core_axis_name="c", subcore_axis_name="s")


@functools.partial(
    pl.kernel,
    mesh=_mesh,
    compiler_params=pltpu.CompilerParams(
        needs_layout_passes=False, use_tc_tiling_on_sc=False),
    out_type=jax.ShapeDtypeStruct((_N_EDGES,), jnp.float32),
    scratch_types=[
        pltpu.VMEM((_E_W,), jnp.int32),
        pltpu.VMEM((_E_W,), jnp.int32),
        pltpu.VMEM((_BPAD, _D // 2), jnp.int32),
        pltpu.VMEM((_BPAD, _D // 2), jnp.int32),
        pltpu.VMEM((_BPAD, _D // 2), jnp.int32),
        pltpu.VMEM((_BPAD, _D // 2), jnp.int32),
        pltpu.VMEM((_BPAD,), jnp.float32),
        pltpu.VMEM((_BPAD,), jnp.float32),
        pltpu.VMEM_SHARED((_N_NODES, _D // 2), jnp.int32),
        pltpu.SemaphoreType.DMA,
        pltpu.SemaphoreType.DMA,
        pltpu.SemaphoreType.DMA,
        pltpu.SemaphoreType.DMA,
    ],
)
def _edge_dist_sc(x_hbm, ei_hbm, out_hbm,
                  idx_i, idx_j, ri0, ri1, rj0, rj1, d0, d1, x_sp,
                  sg0, sg1, ss0, ss1):
    sid = lax.axis_index("s")
    wid = sid * _NC + lax.axis_index("c")
    base_w = wid * _E_W

    rows_per_tile = _N_NODES // _NS
    r0 = sid * rows_per_tile
    pltpu.sync_copy(x_hbm.at[pl.ds(r0, rows_per_tile)],
                    x_sp.at[pl.ds(r0, rows_per_tile)])
    plsc.subcore_barrier()
    rows_i = (ri0, ri1)
    rows_j = (rj0, rj1)
    dist = (d0, d1)
    sem_g = (sg0, sg1)
    sem_s = (ss0, ss1)
    lane = lax.iota(jnp.int32, _L)

    pltpu.sync_copy(ei_hbm.at[0, pl.ds(base_w, _E_W)], idx_i)
    pltpu.sync_copy(ei_hbm.at[1, pl.ds(base_w, _E_W)], idx_j)

    def issue_gather(c, s):
        off = c * _B
        pltpu.async_copy(x_sp.at[idx_i.at[pl.ds(off, _B)]],
                         rows_i[s].at[pl.ds(0, _B)], sem_g[s])
        pltpu.async_copy(x_sp.at[idx_j.at[pl.ds(off, _B)]],
                         rows_j[s].at[pl.ds(0, _B)], sem_g[s])

    def wait_gather(s):
        pltpu.make_async_copy(x_hbm.at[pl.ds(0, _B)],
                              rows_i[s].at[pl.ds(0, _B)], sem_g[s]).wait()
        pltpu.make_async_copy(x_hbm.at[pl.ds(0, _B)],
                              rows_j[s].at[pl.ds(0, _B)], sem_g[s]).wait()

    def wait_store(s):
        pltpu.make_async_copy(dist[s].at[pl.ds(0, _B)],
                              out_hbm.at[pl.ds(0, _B)], sem_s[s]).wait()

    def compute_store(c, s):
        ri, rj, dv = rows_i[s], rows_j[s], dist[s]

        def group_body(g, carry2):
            res = jnp.zeros((_L,), jnp.float32)
            for e_in in range(_L):
                e = g * _L + e_in
                acc_bf = None
                for k in range(_D // (2 * _L)):
                    vi = plsc.bitcast(ri[e, pl.ds(k * _L, _L)], jnp.bfloat16)
                    vj = plsc.bitcast(rj[e, pl.ds(k * _L, _L)], jnp.bfloat16)
                    diff = vi - vj
                    sq = diff * diff
                    acc_bf = sq if acc_bf is None else acc_bf + sq
                lo, hi = plsc.unpack(acc_bf, format=plsc.PackFormat.INTERLEAVED)
                s_val = jnp.sum(lo + hi)
                res = jnp.where(lane == e_in, s_val, res)
            dv[pl.ds(g * _L, _L)] = res
            return carry2

        lax.fori_loop(0, _G, group_body, 0, unroll=4)
        pltpu.async_copy(dv.at[pl.ds(0, _B)],
                         out_hbm.at[pl.ds(base_w + c * _B, _B)], sem_s[s])

    def body(c, s, drain_store, lookahead):
        wait_gather(s)
        if drain_store:
            wait_store(s)
        compute_store(c, s)
        if lookahead:
            issue_gather(c + 2, s)

    issue_gather(0, 0)
    issue_gather(1, 1)
    body(0, 0, drain_store=False, lookahead=True)
    body(1, 1, drain_store=False, lookahead=True)

    def pair_body(p, carry):
        c0 = 2 + 2 * p
        body(c0, 0, drain_store=True, lookahead=True)
        body(c0 + 1, 1, drain_store=True, lookahead=True)
        return carry

    _NPAIRS = (_NCHUNK - 4) // 2
    lax.fori_loop(0, _NPAIRS, pair_body, 0)
    for c in range(2 + 2 * _NPAIRS, _NCHUNK):
        body(c, c % 2, drain_store=True, lookahead=(c + 2 < _NCHUNK))
    wait_store(0)
    wait_store(1)


def _norm_body(temp_ref, dist_ref, out_ref):
    scale = jnp.exp(jnp.clip(temp_ref[0, 0], -5.0, 5.0))
    logits = dist_ref[...] * scale
    lo = jnp.min(logits)
    hi = jnp.max(logits)
    out_ref[...] = 1.0 - (logits - lo) / (hi - lo)


def _pack_word(block):
    lo = block[:, : _D // 2].astype(jnp.bfloat16).astype(jnp.float32)
    hi = block[:, _D // 2:].astype(jnp.bfloat16).astype(jnp.float32)
    lo_bits = jax.lax.bitcast_convert_type(lo, jnp.uint32) >> 16
    hi_bits = jax.lax.bitcast_convert_type(hi, jnp.uint32) & jnp.uint32(
        0xFFFF0000)
    return jax.lax.bitcast_convert_type(lo_bits | hi_bits, jnp.int32)


def _pack_body(x_ref, out_ref):
    out_ref[:, : _D // 2] = _pack_word(x_ref[0::2, :])
    out_ref[:, _D // 2:] = _pack_word(x_ref[1::2, :])


def kernel(x, edge_index, temperature):
    xp = pl.pallas_call(
        _pack_body,
        out_shape=jax.ShapeDtypeStruct((_N_NODES // 2, _D), jnp.int32),
        in_specs=[pl.BlockSpec(memory_space=pltpu.VMEM)],
        out_specs=pl.BlockSpec(memory_space=pltpu.VMEM),
    )(x)
    dist = _edge_dist_sc(xp.reshape(_N_NODES, _D // 2), edge_index)
    temp2d = jnp.reshape(temperature.astype(jnp.float32), (1, 1))
    ew = pl.pallas_call(
        _norm_body,
        out_shape=jax.ShapeDtypeStruct((_N_EDGES,), jnp.float32),
        in_specs=[
            pl.BlockSpec(memory_space=pltpu.SMEM),
            pl.BlockSpec(memory_space=pltpu.VMEM),
        ],
        out_specs=pl.BlockSpec(memory_space=pltpu.VMEM),
    )(temp2d, dist)
    return jnp.reshape(ew, (_N_EDGES, 1))


# speedup vs baseline: 1.2336x; 1.2336x over previous
import functools

import jax
import jax.numpy as jnp
from jax import lax
from jax.experimental import pallas as pl
from jax.experimental.pallas import tpu as pltpu
from jax.experimental.pallas import tpu_sc as plsc

_N_NODES = 10000
_N_EDGES = 320000
_D = 128

_NC = 2
_NS = 16
_L = 16
_NW = _NC * _NS
_E_W = _N_EDGES // _NW
_B = 200
_BPAD = 208
_NCHUNK = _E_W // _B
_G = _BPAD // _L

_mesh = plsc.VectorSubcoreMesh(core_axis_name="c", subcore_axis_name="s")


@functools.partial(
    pl.kernel,
    mesh=_mesh,
    compiler_params=pltpu.CompilerParams(
        needs_layout_passes=False, use_tc_tiling_on_sc=False),
    out_type=jax.ShapeDtypeStruct((_N_EDGES,), jnp.float32),
    scratch_types=[
        pltpu.VMEM((_E_W,), jnp.int32),
        pltpu.VMEM((_E_W,), jnp.int32),
        pltpu.VMEM((_BPAD, _D // 2), jnp.int32),
        pltpu.VMEM((_BPAD, _D // 2), jnp.int32),
        pltpu.VMEM((_BPAD, _D // 2), jnp.int32),
        pltpu.VMEM((_BPAD, _D // 2), jnp.int32),
        pltpu.VMEM((_BPAD,), jnp.float32),
        pltpu.VMEM((_BPAD,), jnp.float32),
        pltpu.VMEM_SHARED((_N_NODES, _D // 2), jnp.int32),
        pltpu.SemaphoreType.DMA,
        pltpu.SemaphoreType.DMA,
        pltpu.SemaphoreType.DMA,
        pltpu.SemaphoreType.DMA,
    ],
)
def _edge_dist_sc(x_hbm, ei_hbm, out_hbm,
                  idx_i, idx_j, ri0, ri1, rj0, rj1, d0, d1, x_sp,
                  sg0, sg1, ss0, ss1):
    sid = lax.axis_index("s")
    wid = sid * _NC + lax.axis_index("c")
    base_w = wid * _E_W

    rows_per_tile = _N_NODES // _NS
    r0 = sid * rows_per_tile
    pltpu.sync_copy(x_hbm.at[pl.ds(r0, rows_per_tile)],
                    x_sp.at[pl.ds(r0, rows_per_tile)])
    plsc.subcore_barrier()
    rows_i = (ri0, ri1)
    rows_j = (rj0, rj1)
    dist = (d0, d1)
    sem_g = (sg0, sg1)
    sem_s = (ss0, ss1)
    lane = lax.iota(jnp.int32, _L)

    pltpu.sync_copy(ei_hbm.at[0, pl.ds(base_w, _E_W)], idx_i)
    pltpu.sync_copy(ei_hbm.at[1, pl.ds(base_w, _E_W)], idx_j)

    def issue_gather(c, s):
        off = c * _B
        pltpu.async_copy(x_sp.at[idx_i.at[pl.ds(off, _B)]],
                         rows_i[s].at[pl.ds(0, _B)], sem_g[s])
        pltpu.async_copy(x_sp.at[idx_j.at[pl.ds(off, _B)]],
                         rows_j[s].at[pl.ds(0, _B)], sem_g[s])

    def wait_gather(s):
        pltpu.make_async_copy(x_hbm.at[pl.ds(0, _B)],
                              rows_i[s].at[pl.ds(0, _B)], sem_g[s]).wait()
        pltpu.make_async_copy(x_hbm.at[pl.ds(0, _B)],
                              rows_j[s].at[pl.ds(0, _B)], sem_g[s]).wait()

    def wait_store(s):
        pltpu.make_async_copy(dist[s].at[pl.ds(0, _B)],
                              out_hbm.at[pl.ds(0, _B)], sem_s[s]).wait()

    def compute_store(c, s):
        ri, rj, dv = rows_i[s], rows_j[s], dist[s]

        @plsc.parallel_loop(0, _G, unroll=2)
        def group_body(g):
            res = jnp.zeros((_L,), jnp.float32)
            for e_in in range(_L):
                e = g * _L + e_in
                acc_bf = None
                for k in range(_D // (2 * _L)):
                    vi = plsc.bitcast(ri[e, pl.ds(k * _L, _L)], jnp.bfloat16)
                    vj = plsc.bitcast(rj[e, pl.ds(k * _L, _L)], jnp.bfloat16)
                    diff = vi - vj
                    sq = diff * diff
                    acc_bf = sq if acc_bf is None else acc_bf + sq
                lo, hi = plsc.unpack(acc_bf, format=plsc.PackFormat.INTERLEAVED)
                s_val = jnp.sum(lo + hi)
                res = jnp.where(lane == e_in, s_val, res)
            dv[pl.ds(g * _L, _L)] = res

        pltpu.async_copy(dv.at[pl.ds(0, _B)],
                         out_hbm.at[pl.ds(base_w + c * _B, _B)], sem_s[s])

    def body(c, s, drain_store, lookahead):
        wait_gather(s)
        if drain_store:
            wait_store(s)
        compute_store(c, s)
        if lookahead:
            issue_gather(c + 2, s)

    issue_gather(0, 0)
    issue_gather(1, 1)
    body(0, 0, drain_store=False, lookahead=True)
    body(1, 1, drain_store=False, lookahead=True)

    def pair_body(p, carry):
        c0 = 2 + 2 * p
        body(c0, 0, drain_store=True, lookahead=True)
        body(c0 + 1, 1, drain_store=True, lookahead=True)
        return carry

    _NPAIRS = (_NCHUNK - 4) // 2
    lax.fori_loop(0, _NPAIRS, pair_body, 0)
    for c in range(2 + 2 * _NPAIRS, _NCHUNK):
        body(c, c % 2, drain_store=True, lookahead=(c + 2 < _NCHUNK))
    wait_store(0)
    wait_store(1)


def _norm_body(temp_ref, dist_ref, out_ref):
    scale = jnp.exp(jnp.clip(temp_ref[0, 0], -5.0, 5.0))
    logits = dist_ref[...] * scale
    lo = jnp.min(logits)
    hi = jnp.max(logits)
    out_ref[...] = 1.0 - (logits - lo) / (hi - lo)


def _pack_word(block):
    lo = block[:, : _D // 2].astype(jnp.bfloat16).astype(jnp.float32)
    hi = block[:, _D // 2:].astype(jnp.bfloat16).astype(jnp.float32)
    lo_bits = jax.lax.bitcast_convert_type(lo, jnp.uint32) >> 16
    hi_bits = jax.lax.bitcast_convert_type(hi, jnp.uint32) & jnp.uint32(
        0xFFFF0000)
    return jax.lax.bitcast_convert_type(lo_bits | hi_bits, jnp.int32)


def _pack_body(x_ref, out_ref):
    out_ref[:, : _D // 2] = _pack_word(x_ref[0::2, :])
    out_ref[:, _D // 2:] = _pack_word(x_ref[1::2, :])


def kernel(x, edge_index, temperature):
    xp = pl.pallas_call(
        _pack_body,
        out_shape=jax.ShapeDtypeStruct((_N_NODES // 2, _D), jnp.int32),
        in_specs=[pl.BlockSpec(memory_space=pltpu.VMEM)],
        out_specs=pl.BlockSpec(memory_space=pltpu.VMEM),
    )(x)
    dist = _edge_dist_sc(xp.reshape(_N_NODES, _D // 2), edge_index)
    temp2d = jnp.reshape(temperature.astype(jnp.float32), (1, 1))
    ew = pl.pallas_call(
        _norm_body,
        out_shape=jax.ShapeDtypeStruct((_N_EDGES,), jnp.float32),
        in_specs=[
            pl.BlockSpec(memory_space=pltpu.SMEM),
            pl.BlockSpec(memory_space=pltpu.VMEM),
        ],
        out_specs=pl.BlockSpec(memory_space=pltpu.VMEM),
    )(temp2d, dist)
    return jnp.reshape(ew, (_N_EDGES, 1))


# speedup vs baseline: 1.2927x; 1.0479x over previous
import functools

import jax
import jax.numpy as jnp
from jax import lax
from jax.experimental import pallas as pl
from jax.experimental.pallas import tpu as pltpu
from jax.experimental.pallas import tpu_sc as plsc

_N_NODES = 10000
_N_EDGES = 320000
_D = 128

_NC = 2
_NS = 16
_L = 16
_NW = _NC * _NS
_E_W = _N_EDGES // _NW
_B = 200
_BPAD = 208
_NCHUNK = _E_W // _B
_G = _BPAD // _L

_mesh = plsc.VectorSubcoreMesh(core_axis_name="c", subcore_axis_name="s")


@functools.partial(
    pl.kernel,
    mesh=_mesh,
    compiler_params=pltpu.CompilerParams(
        needs_layout_passes=False, use_tc_tiling_on_sc=False),
    out_type=jax.ShapeDtypeStruct((_N_EDGES,), jnp.float32),
    scratch_types=[
        pltpu.VMEM((_E_W,), jnp.int32),
        pltpu.VMEM((_E_W,), jnp.int32),
        pltpu.VMEM((_BPAD, _D // 2), jnp.int32),
        pltpu.VMEM((_BPAD, _D // 2), jnp.int32),
        pltpu.VMEM((_BPAD, _D // 2), jnp.int32),
        pltpu.VMEM((_BPAD, _D // 2), jnp.int32),
        pltpu.VMEM((_BPAD,), jnp.float32),
        pltpu.VMEM((_BPAD,), jnp.float32),
        pltpu.VMEM_SHARED((_N_NODES, _D // 2), jnp.int32),
        pltpu.SemaphoreType.DMA,
        pltpu.SemaphoreType.DMA,
        pltpu.SemaphoreType.DMA,
        pltpu.SemaphoreType.DMA,
    ],
)
def _edge_dist_sc(x_hbm, ei_hbm, out_hbm,
                  idx_i, idx_j, ri0, ri1, rj0, rj1, d0, d1, x_sp,
                  sg0, sg1, ss0, ss1):
    sid = lax.axis_index("s")
    wid = sid * _NC + lax.axis_index("c")
    base_w = wid * _E_W

    rows_per_tile = _N_NODES // _NS
    r0 = sid * rows_per_tile
    pltpu.sync_copy(x_hbm.at[pl.ds(r0, rows_per_tile)],
                    x_sp.at[pl.ds(r0, rows_per_tile)])
    plsc.subcore_barrier()
    rows_i = (ri0, ri1)
    rows_j = (rj0, rj1)
    dist = (d0, d1)
    sem_g = (sg0, sg1)
    sem_s = (ss0, ss1)
    lane = lax.iota(jnp.int32, _L)

    pltpu.sync_copy(ei_hbm.at[0, pl.ds(base_w, _E_W)], idx_i)
    pltpu.sync_copy(ei_hbm.at[1, pl.ds(base_w, _E_W)], idx_j)

    def issue_gather(c, s):
        off = c * _B
        pltpu.async_copy(x_sp.at[idx_i.at[pl.ds(off, _B)]],
                         rows_i[s].at[pl.ds(0, _B)], sem_g[s])
        pltpu.async_copy(x_sp.at[idx_j.at[pl.ds(off, _B)]],
                         rows_j[s].at[pl.ds(0, _B)], sem_g[s])

    def wait_gather(s):
        pltpu.make_async_copy(x_hbm.at[pl.ds(0, _B)],
                              rows_i[s].at[pl.ds(0, _B)], sem_g[s]).wait()
        pltpu.make_async_copy(x_hbm.at[pl.ds(0, _B)],
                              rows_j[s].at[pl.ds(0, _B)], sem_g[s]).wait()

    def wait_store(s):
        pltpu.make_async_copy(dist[s].at[pl.ds(0, _B)],
                              out_hbm.at[pl.ds(0, _B)], sem_s[s]).wait()

    def compute_store(c, s):
        ri, rj, dv = rows_i[s], rows_j[s], dist[s]

        def group_body(g, carry2):
            res = jnp.zeros((_L,), jnp.float32)
            for e_in in range(_L):
                e = g * _L + e_in
                acc_bf = None
                for k in range(_D // (2 * _L)):
                    vi = plsc.bitcast(ri[e, pl.ds(k * _L, _L)], jnp.bfloat16)
                    vj = plsc.bitcast(rj[e, pl.ds(k * _L, _L)], jnp.bfloat16)
                    diff = vi - vj
                    sq = diff * diff
                    acc_bf = sq if acc_bf is None else acc_bf + sq
                lo, hi = plsc.unpack(acc_bf, format=plsc.PackFormat.INTERLEAVED)
                s_val = jnp.sum(lo + hi)
                res = jnp.where(lane == e_in, s_val, res)
            dv[pl.ds(g * _L, _L)] = res
            return carry2

        lax.fori_loop(0, _G, group_body, 0, unroll=2)
        pltpu.async_copy(dv.at[pl.ds(0, _B)],
                         out_hbm.at[pl.ds(base_w + c * _B, _B)], sem_s[s])

    def body(c, s, drain_store, lookahead):
        wait_gather(s)
        if drain_store:
            wait_store(s)
        compute_store(c, s)
        if lookahead:
            issue_gather(c + 2, s)

    issue_gather(0, 0)
    issue_gather(1, 1)
    body(0, 0, drain_store=False, lookahead=True)
    body(1, 1, drain_store=False, lookahead=True)

    def pair_body(p, carry):
        c0 = 2 + 2 * p
        body(c0, 0, drain_store=True, lookahead=True)
        body(c0 + 1, 1, drain_store=True, lookahead=True)
        return carry

    _NPAIRS = (_NCHUNK - 4) // 2
    lax.fori_loop(0, _NPAIRS, pair_body, 0)
    for c in range(2 + 2 * _NPAIRS, _NCHUNK):
        body(c, c % 2, drain_store=True, lookahead=(c + 2 < _NCHUNK))
    wait_store(0)
    wait_store(1)


def _norm_body(temp_ref, dist_ref, out_ref):
    scale = jnp.exp(jnp.clip(temp_ref[0, 0], -5.0, 5.0))
    logits = dist_ref[...] * scale
    lo = jnp.min(logits)
    hi = jnp.max(logits)
    out_ref[...] = 1.0 - (logits - lo) / (hi - lo)


def _pack_word(block):
    lo = block[:, : _D // 2].astype(jnp.bfloat16).astype(jnp.float32)
    hi = block[:, _D // 2:].astype(jnp.bfloat16).astype(jnp.float32)
    lo_bits = jax.lax.bitcast_convert_type(lo, jnp.uint32) >> 16
    hi_bits = jax.lax.bitcast_convert_type(hi, jnp.uint32) & jnp.uint32(
        0xFFFF0000)
    return jax.lax.bitcast_convert_type(lo_bits | hi_bits, jnp.int32)


def _pack_body(x_ref, out_ref):
    out_ref[:, : _D // 2] = _pack_word(x_ref[0::2, :])
    out_ref[:, _D // 2:] = _pack_word(x_ref[1::2, :])


def kernel(x, edge_index, temperature):
    xp = pl.pallas_call(
        _pack_body,
        out_shape=jax.ShapeDtypeStruct((_N_NODES // 2, _D), jnp.int32),
        in_specs=[pl.BlockSpec(memory_space=pltpu.VMEM)],
        out_specs=pl.BlockSpec(memory_space=pltpu.VMEM),
    )(x)
    dist = _edge_dist_sc(xp.reshape(_N_NODES, _D // 2), edge_index)
    temp2d = jnp.reshape(temperature.astype(jnp.float32), (1, 1))
    ew = pl.pallas_call(
        _norm_body,
        out_shape=jax.ShapeDtypeStruct((_N_EDGES,), jnp.float32),
        in_specs=[
            pl.BlockSpec(memory_space=pltpu.SMEM),
            pl.BlockSpec(memory_space=pltpu.VMEM),
        ],
        out_specs=pl.BlockSpec(memory_space=pltpu.VMEM),
    )(temp2d, dist)
    return jnp.reshape(ew, (_N_EDGES, 1))
